# Initial kernel scaffold; baseline (speedup 1.0000x reference)
import jax
import jax.numpy as jnp
from jax.experimental import pallas as pl


def kernel(agent_positions, room_params, wall_density):
    raise NotImplementedError("write your pallas kernel here")


# speedup vs baseline: 14.0723x; 14.0723x over previous
import functools

import jax
import jax.numpy as jnp
from jax import lax
from jax.experimental import pallas as pl
from jax.experimental.pallas import tpu as pltpu
from jax.experimental.pallas import tpu_sc as plsc

RES = 256
NBINS = RES * RES
NUM_ROOMS = 16
N_AGENTS = 200000

NUM_CORES = 2
NUM_SUBCORES = 16
NW = NUM_CORES * NUM_SUBCORES
ROWS = 49
CHUNK = ROWS * 128
PAD_N = NW * CHUNK
PAD_COUNT = PAD_N - N_AGENTS
ZCH = NBINS // NUM_SUBCORES


def _sc_histogram(pos_t):
    mesh = plsc.VectorSubcoreMesh(core_axis_name="c", subcore_axis_name="s")

    @functools.partial(
        pl.kernel,
        out_type=jax.ShapeDtypeStruct((NUM_CORES, NBINS), jnp.float32),
        mesh=mesh,
        scratch_types=[
            pltpu.VMEM((CHUNK,), jnp.float32),
            pltpu.VMEM((CHUNK,), jnp.float32),
            pltpu.VMEM((ROWS, 128), jnp.int32),
            pltpu.VMEM((128,), jnp.float32),
            pltpu.VMEM((ZCH,), jnp.float32),
            pltpu.VMEM_SHARED((NBINS,), jnp.float32),
            pltpu.SemaphoreType.DMA,
        ],
    )
    def hist_kernel(pos_hbm, out_hbm, x_v, y_v, idx_v, ones_v, zero_v,
                    hist_sh, sem):
        cid = lax.axis_index("c")
        sid = lax.axis_index("s")
        wid = sid * NUM_CORES + cid
        base = wid * CHUNK

        cp_x = pltpu.make_async_copy(pos_hbm.at[0, pl.ds(base, CHUNK)], x_v, sem)
        cp_y = pltpu.make_async_copy(pos_hbm.at[1, pl.ds(base, CHUNK)], y_v, sem)
        cp_x.start()
        cp_y.start()

        @pl.loop(0, 128, step=16)
        def _(i):
            ones_v[pl.ds(i, 16)] = jnp.full((16,), 1.0, jnp.float32)

        @pl.loop(0, ZCH, step=16)
        def _(i):
            zero_v[pl.ds(i, 16)] = jnp.zeros((16,), jnp.float32)

        pltpu.sync_copy(zero_v, hist_sh.at[pl.ds(sid * ZCH, ZCH)])
        plsc.subcore_barrier()

        cp_x.wait()
        cp_y.wait()

        @pl.loop(0, ROWS)
        def _(r):
            for c in range(8):
                off = r * 128 + c * 16
                x = x_v[pl.ds(off, 16)]
                y = y_v[pl.ds(off, 16)]
                ix = (x * 256.0).astype(jnp.int32)
                iy = (y * 256.0).astype(jnp.int32)
                ix = jnp.minimum(jnp.maximum(ix, 0), RES - 1)
                iy = jnp.minimum(jnp.maximum(iy, 0), RES - 1)
                idx_v[r, pl.ds(c * 16, 16)] = ix * RES + iy

        @pl.loop(0, ROWS)
        def _(r):
            pltpu.sync_copy(ones_v, hist_sh.at[idx_v.at[r]], add=True)

        plsc.subcore_barrier()

        pltpu.sync_copy(hist_sh.at[pl.ds(sid * ZCH, ZCH)],
                        out_hbm.at[cid, pl.ds(sid * ZCH, ZCH)])

    return hist_kernel(pos_t)


def _layout_body(rp_ref, wall_ref, out_ref):
    r = pl.program_id(0)
    cx = rp_ref[r, 0]
    cy = rp_ref[r, 1]
    sx = rp_ref[r, 2]
    sy = rp_ref[r, 3]
    xi = lax.broadcasted_iota(jnp.int32, (RES, RES), 0).astype(jnp.float32) * (
        1.0 / (RES - 1))
    yj = lax.broadcasted_iota(jnp.int32, (RES, RES), 1).astype(jnp.float32) * (
        1.0 / (RES - 1))
    dx = xi - cx
    dy = yj - cy
    e = jnp.exp(-(dx * dx / (2.0 * sx * sx) + dy * dy / (2.0 * sy * sy)))
    out_ref[0] = e * (1.0 - wall_ref[...])


def _tc_layout(room_params, wall_density):
    return pl.pallas_call(
        _layout_body,
        grid=(NUM_ROOMS,),
        in_specs=[
            pl.BlockSpec(memory_space=pltpu.SMEM),
            pl.BlockSpec((RES, RES), lambda r: (0, 0)),
        ],
        out_specs=pl.BlockSpec((1, RES, RES), lambda r: (r, 0, 0)),
        out_shape=jax.ShapeDtypeStruct((NUM_ROOMS, RES, RES), jnp.float32),
    )(room_params, wall_density)


def _flow_body(p_ref, out_ref):
    h = p_ref[0] + p_ref[1]
    ii = lax.broadcasted_iota(jnp.int32, (RES, RES), 0)
    jj = lax.broadcasted_iota(jnp.int32, (RES, RES), 1)
    pad = jnp.where((ii == 0) & (jj == 0), jnp.float32(PAD_COUNT), 0.0)
    h = h - pad
    m = jnp.max(h)
    out_ref[...] = h / (m + 1e-6)


def _tc_flow(partial):
    return pl.pallas_call(
        _flow_body,
        out_shape=jax.ShapeDtypeStruct((RES, RES), jnp.float32),
    )(partial)


def kernel(agent_positions, room_params, wall_density):
    pos_t = jnp.pad(agent_positions.T, ((0, 0), (0, PAD_N - N_AGENTS)))
    partial = _sc_histogram(pos_t)
    dynamic_layout = _tc_layout(room_params, wall_density)
    flow_field = _tc_flow(partial.reshape(NUM_CORES, RES, RES))
    return dynamic_layout, flow_field


# speedup vs baseline: 14.3550x; 1.0201x over previous
import functools

import jax
import jax.numpy as jnp
from jax import lax
from jax.experimental import pallas as pl
from jax.experimental.pallas import tpu as pltpu
from jax.experimental.pallas import tpu_sc as plsc

RES = 256
NBINS = RES * RES
NUM_ROOMS = 16
N_AGENTS = 200000

NUM_CORES = 2
NUM_SUBCORES = 16
NW = NUM_CORES * NUM_SUBCORES
ROWS = 49
CHUNK = ROWS * 128
PAD_N = NW * CHUNK
PAD_COUNT = PAD_N - N_AGENTS
ZCH = NBINS // NUM_SUBCORES


def _sc_histogram(pos_t):
    mesh = plsc.VectorSubcoreMesh(core_axis_name="c", subcore_axis_name="s")

    @functools.partial(
        pl.kernel,
        out_type=jax.ShapeDtypeStruct((NUM_CORES, NBINS), jnp.float32),
        mesh=mesh,
        scratch_types=[
            pltpu.VMEM((CHUNK,), jnp.float32),
            pltpu.VMEM((CHUNK,), jnp.float32),
            pltpu.VMEM((ROWS, 128), jnp.int32),
            pltpu.VMEM((128,), jnp.float32),
            pltpu.VMEM((ZCH,), jnp.float32),
            pltpu.VMEM_SHARED((NBINS,), jnp.float32),
            pltpu.SemaphoreType.DMA,
            pltpu.SemaphoreType.DMA,
        ],
    )
    def hist_kernel(pos_hbm, out_hbm, x_v, y_v, idx_v, ones_v, zero_v,
                    hist_sh, sem, sem_sc):
        cid = lax.axis_index("c")
        sid = lax.axis_index("s")
        wid = sid * NUM_CORES + cid
        base = wid * CHUNK

        cp_x = pltpu.make_async_copy(pos_hbm.at[0, pl.ds(base, CHUNK)], x_v, sem)
        cp_y = pltpu.make_async_copy(pos_hbm.at[1, pl.ds(base, CHUNK)], y_v, sem)
        cp_x.start()
        cp_y.start()

        @pl.loop(0, 128, step=16)
        def _(i):
            ones_v[pl.ds(i, 16)] = jnp.full((16,), 1.0, jnp.float32)

        @pl.loop(0, ZCH, step=16)
        def _(i):
            zero_v[pl.ds(i, 16)] = jnp.zeros((16,), jnp.float32)

        pltpu.sync_copy(zero_v, hist_sh.at[pl.ds(sid * ZCH, ZCH)])
        plsc.subcore_barrier()

        cp_x.wait()
        cp_y.wait()

        @pl.loop(0, ROWS)
        def _(r):
            for c in range(8):
                off = r * 128 + c * 16
                x = x_v[pl.ds(off, 16)]
                y = y_v[pl.ds(off, 16)]
                ix = (x * 256.0).astype(jnp.int32)
                iy = (y * 256.0).astype(jnp.int32)
                ix = jnp.minimum(jnp.maximum(ix, 0), RES - 1)
                iy = jnp.minimum(jnp.maximum(iy, 0), RES - 1)
                idx_v[r, pl.ds(c * 16, 16)] = ix * RES + iy
            pltpu.async_copy(ones_v, hist_sh.at[idx_v.at[r]], sem_sc,
                             add=True)

        @pl.loop(0, ROWS)
        def _(r):
            pltpu.make_async_copy(ones_v, hist_sh.at[idx_v.at[r]],
                                  sem_sc).wait()

        plsc.subcore_barrier()

        pltpu.sync_copy(hist_sh.at[pl.ds(sid * ZCH, ZCH)],
                        out_hbm.at[cid, pl.ds(sid * ZCH, ZCH)])

    return hist_kernel(pos_t)


def _layout_body(rp_ref, wall_ref, out_ref):
    r = pl.program_id(0)
    cx = rp_ref[r, 0]
    cy = rp_ref[r, 1]
    sx = rp_ref[r, 2]
    sy = rp_ref[r, 3]
    xi = lax.broadcasted_iota(jnp.int32, (RES, RES), 0).astype(jnp.float32) * (
        1.0 / (RES - 1))
    yj = lax.broadcasted_iota(jnp.int32, (RES, RES), 1).astype(jnp.float32) * (
        1.0 / (RES - 1))
    dx = xi - cx
    dy = yj - cy
    e = jnp.exp(-(dx * dx / (2.0 * sx * sx) + dy * dy / (2.0 * sy * sy)))
    out_ref[0] = e * (1.0 - wall_ref[...])


def _tc_layout(room_params, wall_density):
    return pl.pallas_call(
        _layout_body,
        grid=(NUM_ROOMS,),
        in_specs=[
            pl.BlockSpec(memory_space=pltpu.SMEM),
            pl.BlockSpec((RES, RES), lambda r: (0, 0)),
        ],
        out_specs=pl.BlockSpec((1, RES, RES), lambda r: (r, 0, 0)),
        out_shape=jax.ShapeDtypeStruct((NUM_ROOMS, RES, RES), jnp.float32),
    )(room_params, wall_density)


def _flow_body(p_ref, out_ref):
    h = p_ref[0] + p_ref[1]
    ii = lax.broadcasted_iota(jnp.int32, (RES, RES), 0)
    jj = lax.broadcasted_iota(jnp.int32, (RES, RES), 1)
    pad = jnp.where((ii == 0) & (jj == 0), jnp.float32(PAD_COUNT), 0.0)
    h = h - pad
    m = jnp.max(h)
    out_ref[...] = h / (m + 1e-6)


def _tc_flow(partial):
    return pl.pallas_call(
        _flow_body,
        out_shape=jax.ShapeDtypeStruct((RES, RES), jnp.float32),
    )(partial)


def kernel(agent_positions, room_params, wall_density):
    pos_t = jnp.pad(agent_positions.T, ((0, 0), (0, PAD_N - N_AGENTS)))
    partial = _sc_histogram(pos_t)
    dynamic_layout = _tc_layout(room_params, wall_density)
    flow_field = _tc_flow(partial.reshape(NUM_CORES, RES, RES))
    return dynamic_layout, flow_field
